# Initial kernel scaffold; baseline (speedup 1.0000x reference)
import jax
import jax.numpy as jnp
from jax.experimental import pallas as pl


def kernel(x, conv_w, conv_b):
    raise NotImplementedError("write your pallas kernel here")


# speedup vs baseline: 1.1916x; 1.1916x over previous
import functools

import jax
import jax.numpy as jnp
from jax.experimental import pallas as pl

B, H, W_, CIN, COUT = 1024, 28, 28, 1, 32
BT = 32


def _conv_body(m_ref, e_ref, b_ref, xp_ref, out_ref):
    xp = xp_ref[...]
    r = BT * H
    acc = jnp.zeros((r, H * COUT), dtype=jnp.float32)
    for dy in range(3):
        x_dy = xp[:, dy:dy + H, :].reshape(r, W_ + 2)
        acc = acc + jnp.dot(x_dy, m_ref[dy],
                            preferred_element_type=jnp.float32)
    xrep = jnp.dot(xp[:, 1:1 + H, :].reshape(r, W_ + 2), e_ref[...],
                   preferred_element_type=jnp.float32)
    out_ref[...] = jnp.where(xrep != 0.0, acc + b_ref[...], 0.0)


@jax.jit
def kernel(x, conv_w, conv_b):
    xsq = x.reshape(B, H, W_)
    xp = jnp.pad(xsq, ((0, 0), (1, 1), (1, 1)))

    sel = jnp.stack([jnp.eye(W_, W_ + 2, k=dx, dtype=jnp.float32)
                     for dx in range(3)])
    m = jnp.einsum("xwj,yxc->ywjc", sel, conv_w[:, :, 0, :])
    m = m.transpose(0, 2, 1, 3).reshape(3, W_ + 2, H * COUT)
    e = jnp.repeat(sel[1], COUT, axis=0).T.reshape(W_ + 2, H * COUT)
    e = jnp.asarray(e, jnp.float32)
    bias = jnp.tile(conv_b, W_).reshape(1, H * COUT)

    grid = B // BT
    out = pl.pallas_call(
        _conv_body,
        grid=(grid,),
        in_specs=[
            pl.BlockSpec((3, W_ + 2, H * COUT), lambda i: (0, 0, 0)),
            pl.BlockSpec((W_ + 2, H * COUT), lambda i: (0, 0)),
            pl.BlockSpec((1, H * COUT), lambda i: (0, 0)),
            pl.BlockSpec((BT, H + 2, W_ + 2), lambda i: (i, 0, 0)),
        ],
        out_specs=pl.BlockSpec((BT * H, H * COUT), lambda i: (i, 0)),
        out_shape=jax.ShapeDtypeStruct((B * H, H * COUT), jnp.float32),
    )(m, e, bias, xp)
    return out.reshape(B * H * W_, COUT)


# speedup vs baseline: 1.2118x; 1.0170x over previous
import jax
import jax.numpy as jnp
from jax.experimental import pallas as pl

B, H, W_, CIN, COUT = 1024, 28, 28, 1, 32
BT = 32
KDIM = 91
NOUT = H * COUT


def _conv_body(m_ref, xp_ref, out_ref):
    xp = xp_ref[...]
    r = BT * H
    slabs = [xp[:, dy:dy + H, :].reshape(r, W_ + 2) for dy in range(3)]
    slabs.append(jnp.ones((r, 1), dtype=jnp.float32))
    xall = jnp.concatenate(slabs, axis=1).astype(jnp.bfloat16)
    res = jnp.dot(xall, m_ref[...], preferred_element_type=jnp.float32)
    acc = res[:, :NOUT]
    xrep = res[:, NOUT:]
    out_ref[...] = jnp.where(xrep != 0.0, acc, 0.0)


@jax.jit
def kernel(x, conv_w, conv_b):
    xsq = x.reshape(B, H, W_)
    xp = jnp.pad(xsq, ((0, 0), (1, 1), (1, 1)))

    sel = jnp.stack([jnp.eye(W_, W_ + 2, k=dx, dtype=jnp.float32)
                     for dx in range(3)])
    m = jnp.einsum("xwj,yxc->ywjc", sel, conv_w[:, :, 0, :])
    m = m.transpose(0, 2, 1, 3).reshape(3 * (W_ + 2), NOUT)
    bias_row = jnp.tile(conv_b, W_).reshape(1, NOUT)
    m = jnp.concatenate([m, bias_row], axis=0)
    e = jnp.repeat(sel[1], COUT, axis=0).T
    e = jnp.concatenate([jnp.zeros((30, NOUT)), e,
                         jnp.zeros((31, NOUT))], axis=0)
    mfull = jnp.concatenate([m, e], axis=1).astype(jnp.bfloat16)

    grid = B // BT
    out = pl.pallas_call(
        _conv_body,
        grid=(grid,),
        in_specs=[
            pl.BlockSpec((KDIM, 2 * NOUT), lambda i: (0, 0)),
            pl.BlockSpec((BT, H + 2, W_ + 2), lambda i: (i, 0, 0)),
        ],
        out_specs=pl.BlockSpec((BT * H, NOUT), lambda i: (i, 0)),
        out_shape=jax.ShapeDtypeStruct((B * H, NOUT), jnp.float32),
    )(mfull, xp)
    return out.reshape(B * H * W_, COUT)


# speedup vs baseline: 1.2387x; 1.0222x over previous
import jax
import jax.numpy as jnp
from jax.experimental import pallas as pl

B, H, W_, CIN, COUT = 1024, 28, 28, 1, 32
BT = 32
KDIM = 91
NOUT = H * COUT


def _conv_body(m_ref, x_ref, out_ref):
    xb = x_ref[...]
    xp = jnp.pad(xb, ((0, 0), (1, 1), (1, 1)))
    r = BT * H
    slabs = [xp[:, dy:dy + H, :].reshape(r, W_ + 2) for dy in range(3)]
    slabs.append(jnp.ones((r, 1), dtype=jnp.float32))
    xall = jnp.concatenate(slabs, axis=1).astype(jnp.bfloat16)
    res = jnp.dot(xall, m_ref[...], preferred_element_type=jnp.float32)
    acc = res[:, :NOUT]
    xrep = res[:, NOUT:]
    out_ref[...] = jnp.where(xrep != 0.0, acc, 0.0)


@jax.jit
def kernel(x, conv_w, conv_b):
    xsq = x.reshape(B, H, W_)

    sel = jnp.stack([jnp.eye(W_, W_ + 2, k=dx, dtype=jnp.float32)
                     for dx in range(3)])
    m = jnp.einsum("xwj,yxc->ywjc", sel, conv_w[:, :, 0, :])
    m = m.transpose(0, 2, 1, 3).reshape(3 * (W_ + 2), NOUT)
    bias_row = jnp.tile(conv_b, W_).reshape(1, NOUT)
    m = jnp.concatenate([m, bias_row], axis=0)
    e = jnp.repeat(sel[1], COUT, axis=0).T
    e = jnp.concatenate([jnp.zeros((30, NOUT)), e,
                         jnp.zeros((31, NOUT))], axis=0)
    mfull = jnp.concatenate([m, e], axis=1).astype(jnp.bfloat16)

    grid = B // BT
    out = pl.pallas_call(
        _conv_body,
        grid=(grid,),
        in_specs=[
            pl.BlockSpec((KDIM, 2 * NOUT), lambda i: (0, 0)),
            pl.BlockSpec((BT, H, W_), lambda i: (i, 0, 0)),
        ],
        out_specs=pl.BlockSpec((BT * H, NOUT), lambda i: (i, 0)),
        out_shape=jax.ShapeDtypeStruct((B * H, NOUT), jnp.float32),
    )(mfull, xsq)
    return out.reshape(B * H * W_, COUT)
